# baseline (speedup 1.0000x reference)
import dataclasses
import functools

import jax
import jax.numpy as jnp
from jax import lax
from jax.experimental import pallas as pl
from jax.experimental.pallas import tpu as pltpu
from jax.experimental.pallas import tpu_sc as plsc

_ALPHA = 0.5
_B = 16384
_N = 1000

_R_SC = _B
_R_TC = 0

_NC = 2
_NS = 16
_NW = _NC * _NS
_RPW = _R_SC // _NW
_CH = 16
_NCHUNK = _RPW // _CH
_SLICES = (_N + 15) // 16
_LAST_BASE = _N - 16
_NEG = -3.4e38


def _sc_rows_kernel(y_hbm, yh_hbm, z_out, part_out,
                    ybuf0, yhbuf0, ybuf1, yhbuf1, zbuf, accbuf,
                    semy0, semh0, semy1, semh1):
    cid = lax.axis_index("core")
    sid = lax.axis_index("subcore")
    wid = sid * _NC + cid
    row0 = _R_TC + wid * _RPW
    iota16 = lax.broadcasted_iota(jnp.int32, (16,), 0)
    last_new = iota16 >= (16 - (_N - (_SLICES - 1) * 16))

    def start_load(c, ybuf, yhbuf, semy, semh):
        r0 = row0 + c * _CH
        pltpu.make_async_copy(y_hbm.at[pl.ds(r0, _CH)], ybuf, semy).start()
        pltpu.make_async_copy(yh_hbm.at[pl.ds(r0, _CH)], yhbuf, semh).start()

    def wait_load(c, ybuf, yhbuf, semy, semh):
        r0 = row0 + c * _CH
        pltpu.make_async_copy(y_hbm.at[pl.ds(r0, _CH)], ybuf, semy).wait()
        pltpu.make_async_copy(yh_hbm.at[pl.ds(r0, _CH)], yhbuf, semh).wait()

    def process_chunk(c, carry, ybuf, yhbuf):
        acc_pick, acc_m, acc_rs = carry

        def row_body(r, rcarry):
            vz_rows, a_pick, a_m, a_rs = rcarry

            vmax = jnp.full((16,), _NEG, jnp.float32)
            vpos = jnp.zeros((16,), jnp.int32)
            for s in range(_SLICES):
                base = s * 16 if s < _SLICES - 1 else _LAST_BASE
                v = ybuf[r, pl.ds(base, 16)]
                if s == _SLICES - 1:
                    v = jnp.where(last_new, v, _NEG)
                gt = v > vmax
                vpos = jnp.where(gt, base, vpos)
                vmax = jnp.where(gt, v, vmax)
            rowmax = jnp.max(vmax)
            cand = jnp.where(vmax == rowmax, vpos + iota16, jnp.int32(2**30))
            j = jnp.min(cand)

            hmax = jnp.full((16,), _NEG, jnp.float32)
            hsum = jnp.zeros((16,), jnp.float32)
            for s in range(_SLICES):
                base = s * 16 if s < _SLICES - 1 else _LAST_BASE
                v = yhbuf[r, pl.ds(base, 16)]
                if s == _SLICES - 1:
                    hmax = jnp.maximum(hmax, jnp.where(last_new, v, _NEG))
                    hsum = hsum + jnp.where(last_new, v, 0.0)
                else:
                    hmax = jnp.maximum(hmax, v)
                    hsum = hsum + v
            m = jnp.max(hmax)
            rs = jnp.sum(hsum)

            ez = jnp.zeros((16,), jnp.float32)
            for s in range(_SLICES):
                base = s * 16 if s < _SLICES - 1 else _LAST_BASE
                v = yhbuf[r, pl.ds(base, 16)]
                e = jnp.exp(v - m)
                if s == _SLICES - 1:
                    e = jnp.where(last_new, e, 0.0)
                ez = ez + e
            z = jnp.sum(ez)

            p0 = jnp.minimum(j - lax.rem(j, 16), _LAST_BASE)
            pv = yhbuf[r, pl.ds(p0, 16)]
            pick = jnp.sum(jnp.where(iota16 == (j - p0), pv, 0.0))

            vz_rows = jnp.where(iota16 == r, z, vz_rows)
            return (vz_rows, a_pick + pick, a_m + m, a_rs + rs)

        vz_rows, acc_pick, acc_m, acc_rs = lax.fori_loop(
            0, _CH, row_body,
            (jnp.zeros((16,), jnp.float32), acc_pick, acc_m, acc_rs),
        )
        zbuf[...] = vz_rows
        pltpu.sync_copy(zbuf, z_out.at[wid, pl.ds(c * _CH, _CH)])
        return (acc_pick, acc_m, acc_rs)

    start_load(0, ybuf0, yhbuf0, semy0, semh0)

    def pair_body(c2, carry):
        c = 2 * c2
        wait_load(c, ybuf0, yhbuf0, semy0, semh0)
        start_load(c + 1, ybuf1, yhbuf1, semy1, semh1)
        carry = process_chunk(c, carry, ybuf0, yhbuf0)
        wait_load(c + 1, ybuf1, yhbuf1, semy1, semh1)

        @pl.when(c + 2 < _NCHUNK)
        def _():
            start_load(c + 2, ybuf0, yhbuf0, semy0, semh0)

        return process_chunk(c + 1, carry, ybuf1, yhbuf1)

    acc_pick, acc_m, acc_rs = lax.fori_loop(
        0, _NCHUNK // 2, pair_body,
        (jnp.float32(0.0), jnp.float32(0.0), jnp.float32(0.0)),
    )
    accbuf[...] = jnp.where(
        iota16 == 0, acc_pick,
        jnp.where(iota16 == 1, acc_m, jnp.where(iota16 == 2, acc_rs, 0.0)),
    )
    pltpu.sync_copy(accbuf, part_out.at[wid])


def _tc_combine_kernel(part_ref, z_ref, sup_ref, out_ref):
    a = sup_ref[1, 0]
    d = sup_ref[0, 0]
    c1 = _ALPHA + (1.0 - _ALPHA) * (d - a)
    c2 = (1.0 - _ALPHA) * a

    parts = part_ref[...]
    lane = lax.broadcasted_iota(jnp.int32, parts.shape, 1)
    sum_pick = jnp.sum(jnp.where(lane == 0, parts, 0.0))
    sum_m = jnp.sum(jnp.where(lane == 1, parts, 0.0))
    sum_rs = jnp.sum(jnp.where(lane == 2, parts, 0.0))
    sum_logz = jnp.sum(jnp.log(z_ref[...]))
    sum_shift = sum_m + sum_logz

    p_sc = c1 * (sum_pick - sum_shift) + c2 * (sum_rs - jnp.float32(_N) * sum_shift)
    loss = -p_sc * (1.0 / _B)
    out_ref[...] = loss.reshape(1, 1)


_sc_mesh = plsc.VectorSubcoreMesh(core_axis_name="core", subcore_axis_name="subcore")

_sc_params = pltpu.CompilerParams()
if "needs_layout_passes" in pltpu.CompilerParams.__dataclass_fields__:
    _sc_params = dataclasses.replace(_sc_params, needs_layout_passes=False)


@functools.partial(
    pl.kernel,
    out_type=[
        jax.ShapeDtypeStruct((_NW, _RPW), jnp.float32),
        jax.ShapeDtypeStruct((_NW, 16), jnp.float32),
    ],
    mesh=_sc_mesh,
    compiler_params=_sc_params,
    scratch_types=[
        pltpu.VMEM((_CH, _N), jnp.float32),
        pltpu.VMEM((_CH, _N), jnp.float32),
        pltpu.VMEM((_CH, _N), jnp.float32),
        pltpu.VMEM((_CH, _N), jnp.float32),
        pltpu.VMEM((16,), jnp.float32),
        pltpu.VMEM((16,), jnp.float32),
        pltpu.SemaphoreType.DMA,
        pltpu.SemaphoreType.DMA,
        pltpu.SemaphoreType.DMA,
        pltpu.SemaphoreType.DMA,
    ],
)
def _sc_kernel(y_hbm, yh_hbm, z_out, part_out,
               ybuf0, yhbuf0, ybuf1, yhbuf1, zbuf, accbuf,
               semy0, semh0, semy1, semh1):
    _sc_rows_kernel(y_hbm, yh_hbm, z_out, part_out,
                    ybuf0, yhbuf0, ybuf1, yhbuf1, zbuf, accbuf,
                    semy0, semh0, semy1, semh1)


@functools.partial(jax.jit, static_argnames=())
def kernel(y_h, y, supervise):
    y_h = y_h.astype(jnp.float32)
    z_sc, parts_sc = _sc_kernel(y, y_h)

    out = pl.pallas_call(
        _tc_combine_kernel,
        in_specs=[
            pl.BlockSpec((_NW, 16), lambda: (0, 0)),
            pl.BlockSpec((_NW, _RPW), lambda: (0, 0)),
            pl.BlockSpec((8, 128), lambda: (0, 0)),
        ],
        out_specs=pl.BlockSpec((1, 1), lambda: (0, 0)),
        out_shape=jax.ShapeDtypeStruct((1, 1), jnp.float32),
    )(parts_sc, z_sc, lax.slice(supervise, (0, 0), (8, 128)))
    return out[0, 0]


# speedup vs baseline: 1.6842x; 1.6842x over previous
import functools

import jax
import jax.numpy as jnp
from jax.experimental import pallas as pl
from jax.experimental.pallas import tpu as pltpu

_ALPHA = 0.5
_B = 16384
_N = 1000
_ROWS = 2048


def _loss_kernel(y_h_ref, y_ref, sup_ref, out_ref):
    step = pl.program_id(0)

    yh = y_h_ref[...]
    yv = y_ref[...]

    m = jnp.max(yh, axis=1)
    z = jnp.sum(jnp.exp(yh - m[:, None]), axis=1)
    shift = m + jnp.log(z)
    rs = jnp.sum(yh, axis=1)
    rowsum_logp = rs - _N * shift

    iota = jax.lax.broadcasted_iota(jnp.int32, yv.shape, 1)
    vmax = jnp.max(yv, axis=1)
    j = jnp.min(jnp.where(yv == vmax[:, None], iota, _N), axis=1)
    pick = jnp.sum(jnp.where(iota == j[:, None], yh, 0.0), axis=1)
    lp_pick = pick - shift

    a = sup_ref[1, 0]
    d = sup_ref[0, 0]
    c1 = _ALPHA + (1.0 - _ALPHA) * (d - a)
    c2 = (1.0 - _ALPHA) * a

    partial = -jnp.sum(c1 * lp_pick + c2 * rowsum_logp) * (1.0 / _B)

    @pl.when(step == 0)
    def _init():
        out_ref[...] = jnp.zeros_like(out_ref)

    out_ref[...] += partial


@functools.partial(jax.jit, static_argnames=())
def kernel(y_h, y, supervise):
    out = pl.pallas_call(
        _loss_kernel,
        grid=(_B // _ROWS,),
        in_specs=[
            pl.BlockSpec((_ROWS, _N), lambda i: (i, 0)),
            pl.BlockSpec((_ROWS, _N), lambda i: (i, 0)),
            pl.BlockSpec((8, 128), lambda i: (0, 0)),
        ],
        out_specs=pl.BlockSpec((1, 1), lambda i: (0, 0)),
        out_shape=jax.ShapeDtypeStruct((1, 1), jnp.float32),
        compiler_params=pltpu.CompilerParams(
            dimension_semantics=("arbitrary",),
        ),
    )(y_h.astype(jnp.float32), y, supervise)
    return out[0, 0]
